# baseline (speedup 1.0000x reference)
import jax
import jax.numpy as jnp
from jax import lax
from jax.experimental import pallas as pl
from jax.experimental.pallas import tpu as pltpu
from jax.experimental.pallas import tpu_sc as plsc

BATCH = 16384
F = 64
NC = 2
NS = 16
NW = NC * NS
BPW = BATCH // NW
G = 16
NGROUP = BPW // G
SLOTS = 4
FU = 8


def _mf_body(user_hbm, item_hbm, utt_hbm, itt_hbm, wb_hbm, bv_hbm, out_hbm,
             uidx_v, iidx_v, ubuf, ibuf, wb_v, bv_v, out_v, *sems):
    wid = lax.axis_index("s") * NC + lax.axis_index("c")
    base = wid * BPW
    lanes = lax.iota(jnp.int32, 16)

    pltpu.sync_copy(user_hbm.at[pl.ds(base, BPW)], uidx_v)
    pltpu.sync_copy(item_hbm.at[pl.ds(base, BPW)], iidx_v)
    pltpu.sync_copy(wb_hbm, wb_v)
    pltpu.sync_copy(bv_hbm, bv_v)

    usems = sems[:SLOTS]
    isems = sems[SLOTS:]

    def fire(slot, ku, ki):
        qu = pl.multiple_of((ku >> 7) << 7, 128)
        qi = pl.multiple_of((ki >> 7) << 7, 128)
        pltpu.async_copy(utt_hbm.at[:, pl.ds(qu, 128)], ubuf.at[slot], usems[slot])
        pltpu.async_copy(itt_hbm.at[:, pl.ds(qi, 128)], ibuf.at[slot], isems[slot])

    def drain(slot):
        pltpu.make_async_copy(utt_hbm.at[:, pl.ds(0, 128)], ubuf.at[slot], usems[slot]).wait()
        pltpu.make_async_copy(itt_hbm.at[:, pl.ds(0, 128)], ibuf.at[slot], isems[slot]).wait()

    kv0u = uidx_v[pl.ds(0, 16)]
    kv0i = iidx_v[pl.ds(0, 16)]
    for j in range(SLOTS):
        fire(j, kv0u[j], kv0i[j])

    def group(g, carry):
        kvu = uidx_v[pl.ds(g * G, 16)]
        kvi = iidx_v[pl.ds(g * G, 16)]
        nxt = ((g + 1) & (NGROUP - 1)) * G
        kvu_n = uidx_v[pl.ds(nxt, 16)]
        kvi_n = iidx_v[pl.ds(nxt, 16)]
        out = bv_v[...]
        for j in range(G):
            slot = j % SLOTS
            ku = kvu[j]
            ki = kvi[j]
            jcu = (ku & 127) & ~15
            jci = (ki & 127) & ~15
            lu = jnp.full((16,), ku & 15, jnp.int32)
            li = jnp.full((16,), ki & 15, jnp.int32)
            drain(slot)
            ub = ubuf.at[slot]
            ib = ibuf.at[slot]

            def fbody(ff, acc):
                f0 = ff * FU
                for df in range(FU):
                    uf = jnp.take(ub[f0 + df, pl.ds(jcu, 16)], lu)
                    vf = jnp.take(ib[f0 + df, pl.ds(jci, 16)], li)
                    acc = acc + (uf * vf) * wb_v[f0 + df]
                return acc

            acc = lax.fori_loop(0, F // FU, fbody, jnp.zeros((16,), jnp.float32))

            if j < G - SLOTS:
                knu, kni = kvu[j + SLOTS], kvi[j + SLOTS]
            else:
                knu, kni = kvu_n[j + SLOTS - G], kvi_n[j + SLOTS - G]

            @pl.when(g * G + j + SLOTS < BPW)
            def _():
                fire(slot, knu, kni)

            out = jnp.where(lanes == j, out + acc, out)
        out_v[pl.ds(g * G, 16)] = 1.0 / (1.0 + jnp.exp(-out))
        return carry

    lax.fori_loop(0, NGROUP, group, 0)
    pltpu.sync_copy(out_v, out_hbm.at[pl.ds(base, BPW)])


@jax.jit
def kernel(user, item, user_table, item_table, W, b):
    user = user.astype(jnp.int32)
    item = item.astype(jnp.int32)
    pu = jnp.argsort(user)
    su = user[pu]
    user = jnp.zeros_like(user).at[pu].set(su)
    pi = jnp.argsort(item)
    si = item[pi]
    item = jnp.zeros_like(item).at[pi].set(si)
    utt = user_table.T
    itt = item_table.T
    wb = jnp.broadcast_to(W.reshape(F, 1), (F, 16))
    bv = jnp.broadcast_to(b.reshape(1), (16,))

    mesh = plsc.VectorSubcoreMesh(core_axis_name="c", subcore_axis_name="s")
    run = pl.kernel(
        _mf_body,
        out_type=jax.ShapeDtypeStruct((BATCH,), jnp.float32),
        mesh=mesh,
        scratch_types=[
            pltpu.VMEM((BPW,), jnp.int32),
            pltpu.VMEM((BPW,), jnp.int32),
            pltpu.VMEM((SLOTS, F, 128), jnp.float32),
            pltpu.VMEM((SLOTS, F, 128), jnp.float32),
            pltpu.VMEM((F, 16), jnp.float32),
            pltpu.VMEM((16,), jnp.float32),
            pltpu.VMEM((BPW,), jnp.float32),
        ] + [pltpu.SemaphoreType.DMA] * (2 * SLOTS),
    )
    return run(user, item, utt, itt, wb, bv)


# speedup vs baseline: 1.3487x; 1.3487x over previous
import jax
import jax.numpy as jnp
from jax import lax
from jax.experimental import pallas as pl
from jax.experimental.pallas import tpu as pltpu
from jax.experimental.pallas import tpu_sc as plsc

BATCH = 16384
F = 64
NC = 2
NS = 16
NW = NC * NS
BPW = BATCH // NW
G = 16
NGROUP = BPW // G
SLOTS = 4
FU = 8


def _mf_body(user_hbm, item_hbm, utt_hbm, itt_hbm, wb_hbm, bv_hbm, out_hbm,
             uidx_v, iidx_v, ubuf, ibuf, wb_v, bv_v, out_v, *sems):
    wid = lax.axis_index("s") * NC + lax.axis_index("c")
    base = wid * BPW
    lanes = lax.iota(jnp.int32, 16)

    pltpu.sync_copy(user_hbm.at[pl.ds(base, BPW)], uidx_v)
    pltpu.sync_copy(item_hbm.at[pl.ds(base, BPW)], iidx_v)
    pltpu.sync_copy(wb_hbm, wb_v)
    pltpu.sync_copy(bv_hbm, bv_v)

    usems = sems[:SLOTS]
    isems = sems[SLOTS:]

    def fire(slot, ku, ki):
        qu = pl.multiple_of((ku >> 7) << 7, 128)
        qi = pl.multiple_of((ki >> 7) << 7, 128)
        pltpu.async_copy(utt_hbm.at[:, pl.ds(qu, 128)], ubuf.at[slot], usems[slot])
        pltpu.async_copy(itt_hbm.at[:, pl.ds(qi, 128)], ibuf.at[slot], isems[slot])

    def drain(slot):
        pltpu.make_async_copy(utt_hbm.at[:, pl.ds(0, 128)], ubuf.at[slot], usems[slot]).wait()
        pltpu.make_async_copy(itt_hbm.at[:, pl.ds(0, 128)], ibuf.at[slot], isems[slot]).wait()

    kv0u = uidx_v[pl.ds(0, 16)]
    kv0i = iidx_v[pl.ds(0, 16)]
    for j in range(SLOTS):
        fire(j, kv0u[j], kv0i[j])

    def group(g, carry):
        kvu = uidx_v[pl.ds(g * G, 16)]
        kvi = iidx_v[pl.ds(g * G, 16)]
        nxt = ((g + 1) & (NGROUP - 1)) * G
        kvu_n = uidx_v[pl.ds(nxt, 16)]
        kvi_n = iidx_v[pl.ds(nxt, 16)]
        out = bv_v[...]
        for j in range(G):
            slot = j % SLOTS
            ku = kvu[j]
            ki = kvi[j]
            jcu = (ku & 127) & ~15
            jci = (ki & 127) & ~15
            lu = jnp.full((16,), ku & 15, jnp.int32)
            li = jnp.full((16,), ki & 15, jnp.int32)
            drain(slot)
            ub = ubuf.at[slot]
            ib = ibuf.at[slot]

            def fbody(ff, acc):
                f0 = ff * FU
                for df in range(FU):
                    uf = jnp.take(ub[f0 + df, pl.ds(jcu, 16)], lu)
                    vf = jnp.take(ib[f0 + df, pl.ds(jci, 16)], li)
                    acc = acc + (uf * vf) * wb_v[f0 + df]
                return acc

            acc = lax.fori_loop(0, F // FU, fbody, jnp.zeros((16,), jnp.float32))

            if j < G - SLOTS:
                knu, kni = kvu[j + SLOTS], kvi[j + SLOTS]
            else:
                knu, kni = kvu_n[j + SLOTS - G], kvi_n[j + SLOTS - G]

            @pl.when(g * G + j + SLOTS < BPW)
            def _():
                fire(slot, knu, kni)

            out = jnp.where(lanes == j, out + acc, out)
        out_v[pl.ds(g * G, 16)] = 1.0 / (1.0 + jnp.exp(-out))
        return carry

    lax.fori_loop(0, NGROUP, group, 0)
    pltpu.sync_copy(out_v, out_hbm.at[pl.ds(base, BPW)])


@jax.jit
def kernel(user, item, user_table, item_table, W, b):
    user = user.astype(jnp.int32)
    item = item.astype(jnp.int32)
    utt = user_table.T
    itt = item_table.T
    wb = jnp.broadcast_to(W.reshape(F, 1), (F, 16))
    bv = jnp.broadcast_to(b.reshape(1), (16,))

    mesh = plsc.VectorSubcoreMesh(core_axis_name="c", subcore_axis_name="s")
    run = pl.kernel(
        _mf_body,
        out_type=jax.ShapeDtypeStruct((BATCH,), jnp.float32),
        mesh=mesh,
        scratch_types=[
            pltpu.VMEM((BPW,), jnp.int32),
            pltpu.VMEM((BPW,), jnp.int32),
            pltpu.VMEM((SLOTS, F, 128), jnp.float32),
            pltpu.VMEM((SLOTS, F, 128), jnp.float32),
            pltpu.VMEM((F, 16), jnp.float32),
            pltpu.VMEM((16,), jnp.float32),
            pltpu.VMEM((BPW,), jnp.float32),
        ] + [pltpu.SemaphoreType.DMA] * (2 * SLOTS),
    )
    return run(user, item, utt, itt, wb, bv)
